# baseline (speedup 1.0000x reference)
import functools

import jax
import jax.numpy as jnp
from jax import lax
from jax.experimental import pallas as pl
from jax.experimental.pallas import tpu as pltpu
from jax.experimental.pallas import tpu_sc as plsc

_VOCAB = 8192
_EMBED = 32
_N = 8192
_IN_DIM = 64
_TN = 512
_TV = 512
_NN = _N // _TN
_NV = _VOCAB // _TV
_DPAD = 128


def _a_body(z_ref, w1_ref, b1_ref, embb_ref, w2_ref, b2_ref, zp_ref, table_ref):
    zp_ref[...] = jnp.dot(z_ref[...], w1_ref[...],
                          preferred_element_type=jnp.float32) + b1_ref[...]
    table_ref[...] = lax.dot_general(
        embb_ref[...], w2_ref[...], (((1,), (0,)), ((), ())),
        preferred_element_type=jnp.float32) + b2_ref[...]


def _zp_and_table(z, embb, W1, b1, W2, b2):
    return pl.pallas_call(
        _a_body,
        grid=(_NN,),
        in_specs=[
            pl.BlockSpec((_TN, _IN_DIM), lambda i: (i, 0)),
            pl.BlockSpec((_IN_DIM, _EMBED), lambda i: (0, 0)),
            pl.BlockSpec((1, _EMBED), lambda i: (0, 0)),
            pl.BlockSpec((_TV, _EMBED), lambda i: (i, 0)),
            pl.BlockSpec((_EMBED, _DPAD), lambda i: (0, 0)),
            pl.BlockSpec((1, _DPAD), lambda i: (0, 0)),
        ],
        out_specs=[
            pl.BlockSpec((_TN, _EMBED), lambda i: (i, 0)),
            pl.BlockSpec((_TV, _DPAD), lambda i: (i, 0)),
        ],
        out_shape=[
            jax.ShapeDtypeStruct((_N, _EMBED), jnp.float32),
            jax.ShapeDtypeStruct((_VOCAB, _DPAD), jnp.float32),
        ],
    )(z, W1, b1.reshape(1, _EMBED), embb,
      jnp.pad(W2, ((0, 0), (0, _DPAD - _IN_DIM))),
      jnp.pad(b2, (0, _DPAD - _IN_DIM)).reshape(1, _DPAD))


def _b_body(zpb_ref, embb_ref, zpn_ref, en_ref, tok_ref, best_s, besti_s):
    v = pl.program_id(0)
    n = pl.program_id(1)
    nds = pl.ds(pl.multiple_of(n * _TN, _TN), _TN)
    vds = pl.ds(pl.multiple_of(v * _TV, _TV), _TV)

    @pl.when(v == 0)
    def _():
        best_s[nds, :] = jnp.full((_TN, 1), jnp.inf, dtype=jnp.float32)
        besti_s[nds, :] = jnp.zeros((_TN, 1), dtype=jnp.int32)

    zpb = zpb_ref[nds, :]
    e = embb_ref[vds, :]
    mm = lax.dot_general(zpb, e, (((1,), (1,)), ((), ())),
                         preferred_element_type=jnp.float32)
    d = (zpn_ref[nds, :] + en_ref[...].reshape(1, _TV)) - 2.0 * mm
    lmin = jnp.min(d, axis=1).reshape(_TN, 1)
    cols = lax.broadcasted_iota(jnp.int32, (_TN, _TV), 1)
    cand = jnp.where(d == lmin, cols, jnp.int32(0x7FFFFFFF))
    lidx = jnp.min(cand, axis=1).reshape(_TN, 1)
    upd = lmin < best_s[nds, :]
    best_s[nds, :] = jnp.where(upd, lmin, best_s[nds, :])
    besti_s[nds, :] = jnp.where(upd, v * _TV + lidx, besti_s[nds, :])

    @pl.when(v % (2048 // _TV) == (2048 // _TV) - 1)
    def _():
        best_s[nds, :] = best_s[nds, :].astype(jnp.bfloat16).astype(jnp.float32)

    @pl.when(v == _NV - 1)
    def _():
        tok_ref[...] = besti_s[nds, :]


def _tokens(zpb, embb, zpn2, en3):
    return pl.pallas_call(
        _b_body,
        grid=(_NV, _NN),
        in_specs=[
            pl.BlockSpec((_N, _EMBED), lambda v, n: (0, 0)),
            pl.BlockSpec((_VOCAB, _EMBED), lambda v, n: (0, 0)),
            pl.BlockSpec((_N, 1), lambda v, n: (0, 0)),
            pl.BlockSpec((1, 1, _TV), lambda v, n: (v, 0, 0)),
        ],
        out_specs=pl.BlockSpec((_TN, 1), lambda v, n: (n, 0)),
        out_shape=jax.ShapeDtypeStruct((_N, 1), jnp.int32),
        scratch_shapes=[
            pltpu.VMEM((_N, 1), jnp.float32),
            pltpu.VMEM((_N, 1), jnp.int32),
        ],
    )(zpb, embb, zpn2, en3)


def _gather_rows(table, idx):
    info = plsc.get_sparse_core_info()
    nw = info.num_cores * info.num_subcores
    b_per_w = _N // nw
    mesh = plsc.VectorSubcoreMesh(core_axis_name="c", subcore_axis_name="s")

    @functools.partial(
        pl.kernel,
        mesh=mesh,
        out_type=jax.ShapeDtypeStruct((_N, _DPAD), jnp.float32),
        scratch_types=[
            pltpu.VMEM((b_per_w,), jnp.int32),
            pltpu.VMEM((b_per_w, _DPAD), jnp.float32),
            pltpu.SemaphoreType.DMA,
        ],
    )
    def k(table_hbm, idx_hbm, out_hbm, idx_v, rows_v, sem):
        wid = lax.axis_index("s") * info.num_cores + lax.axis_index("c")
        base = wid * b_per_w
        pltpu.sync_copy(idx_hbm.at[pl.ds(base, b_per_w)], idx_v)
        pltpu.async_copy(table_hbm.at[idx_v], rows_v, sem).wait()
        pltpu.sync_copy(rows_v, out_hbm.at[pl.ds(base, b_per_w)])

    return k(table, idx)


def kernel(z, emb, W1, b1, W2, b2):
    embb = emb.astype(jnp.bfloat16)
    zp, table = _zp_and_table(z, embb, W1, b1, W2, b2)
    zpb = zp.astype(jnp.bfloat16)
    zpn2 = jnp.sum(zp * zp, axis=1).reshape(_N, 1)
    en3 = jnp.sum(emb * emb, axis=1).reshape(_NV, 1, _TV)
    tokens = _tokens(zpb, embb, zpn2, en3).reshape(_N)
    z_q = _gather_rows(table, tokens)[:, :_IN_DIM]
    return tokens, z_q


# speedup vs baseline: 1.4692x; 1.4692x over previous
import functools

import jax
import jax.numpy as jnp
from jax import lax
from jax.experimental import pallas as pl
from jax.experimental.pallas import tpu as pltpu
from jax.experimental.pallas import tpu_sc as plsc

_VOCAB = 8192
_EMBED = 32
_N = 8192
_IN_DIM = 64
_TN = 512
_TV = 512
_NN = _N // _TN
_NV = _VOCAB // _TV
_DPAD = 128


def _a_body(z_ref, w1_ref, b1_ref, emb_ref, w2_ref, b2_ref,
            zp_ref, zpb_ref, embb_ref, table_ref):
    zp = jnp.dot(z_ref[...], w1_ref[...],
                 preferred_element_type=jnp.float32) + b1_ref[...]
    zp_ref[...] = zp
    zpb_ref[...] = zp.astype(jnp.bfloat16)
    embb = emb_ref[...].astype(jnp.bfloat16)
    embb_ref[...] = embb
    table_ref[...] = lax.dot_general(
        embb, w2_ref[...], (((1,), (0,)), ((), ())),
        preferred_element_type=jnp.float32) + b2_ref[...]


def _zp_and_table(z, emb, W1, b1, W2, b2):
    return pl.pallas_call(
        _a_body,
        grid=(_NN,),
        in_specs=[
            pl.BlockSpec((_TN, _IN_DIM), lambda i: (i, 0)),
            pl.BlockSpec((_IN_DIM, _EMBED), lambda i: (0, 0)),
            pl.BlockSpec((1, _EMBED), lambda i: (0, 0)),
            pl.BlockSpec((_TN, _EMBED), lambda i: (i, 0)),
            pl.BlockSpec((_EMBED, _DPAD), lambda i: (0, 0)),
            pl.BlockSpec((1, _DPAD), lambda i: (0, 0)),
        ],
        out_specs=[
            pl.BlockSpec((_TN, _EMBED), lambda i: (i, 0)),
            pl.BlockSpec((_TN, _EMBED), lambda i: (i, 0)),
            pl.BlockSpec((_TN, _EMBED), lambda i: (i, 0)),
            pl.BlockSpec((_TN, _DPAD), lambda i: (i, 0)),
        ],
        out_shape=[
            jax.ShapeDtypeStruct((_N, _EMBED), jnp.float32),
            jax.ShapeDtypeStruct((_N, _EMBED), jnp.bfloat16),
            jax.ShapeDtypeStruct((_VOCAB, _EMBED), jnp.bfloat16),
            jax.ShapeDtypeStruct((_VOCAB, _DPAD), jnp.float32),
        ],
    )(z, W1, b1.reshape(1, _EMBED), emb,
      jnp.pad(W2, ((0, 0), (0, _DPAD - _IN_DIM))),
      jnp.pad(b2, (0, _DPAD - _IN_DIM)).reshape(1, _DPAD))


_TVB = 2048
_NVB = _VOCAB // _TVB


def _b_body(zpb_ref, embb_ref, zpn_ref, en_ref, tok_ref, best_s, besti_s):
    v = pl.program_id(0)
    n = pl.program_id(1)
    nds = pl.ds(pl.multiple_of(n * _TN, _TN), _TN)
    vds = pl.ds(pl.multiple_of(v * _TVB, _TVB), _TVB)

    @pl.when(v == 0)
    def _():
        best_s[nds, :] = jnp.full((_TN, 1), jnp.inf, dtype=jnp.float32)
        besti_s[nds, :] = jnp.zeros((_TN, 1), dtype=jnp.int32)

    zpb = zpb_ref[nds, :]
    e = embb_ref[vds, :]
    mm = lax.dot_general(zpb, e, (((1,), (1,)), ((), ())),
                         preferred_element_type=jnp.float32)
    d = (zpn_ref[nds, :] + en_ref[...].reshape(1, _TVB)) - 2.0 * mm
    lmin = jnp.min(d, axis=1).reshape(_TN, 1)
    cols = lax.broadcasted_iota(jnp.int32, (_TN, _TVB), 1)
    cand = jnp.where(d == lmin, cols, jnp.int32(0x7FFFFFFF))
    lidx = jnp.min(cand, axis=1).reshape(_TN, 1)
    upd = lmin < best_s[nds, :]
    besti_s[nds, :] = jnp.where(upd, v * _TVB + lidx, besti_s[nds, :])
    best_s[nds, :] = jnp.where(upd, lmin, best_s[nds, :]).astype(
        jnp.bfloat16).astype(jnp.float32)

    @pl.when(v == _NVB - 1)
    def _():
        tok_ref[...] = besti_s[nds, :]


def _tokens(zpb, embb, zpn2, en3):
    return pl.pallas_call(
        _b_body,
        grid=(_NVB, _NN),
        in_specs=[
            pl.BlockSpec((_N, _EMBED), lambda v, n: (0, 0)),
            pl.BlockSpec((_VOCAB, _EMBED), lambda v, n: (0, 0)),
            pl.BlockSpec((_N, 1), lambda v, n: (0, 0)),
            pl.BlockSpec((1, 1, _TVB), lambda v, n: (v, 0, 0)),
        ],
        out_specs=pl.BlockSpec((_TN, 1), lambda v, n: (n, 0)),
        out_shape=jax.ShapeDtypeStruct((_N, 1), jnp.int32),
        scratch_shapes=[
            pltpu.VMEM((_N, 1), jnp.float32),
            pltpu.VMEM((_N, 1), jnp.int32),
        ],
    )(zpb, embb, zpn2, en3)


def _gather_rows(table, idx):
    info = plsc.get_sparse_core_info()
    nw = info.num_cores * info.num_subcores
    b_per_w = _N // nw
    mesh = plsc.VectorSubcoreMesh(core_axis_name="c", subcore_axis_name="s")

    @functools.partial(
        pl.kernel,
        mesh=mesh,
        out_type=jax.ShapeDtypeStruct((_N, _DPAD), jnp.float32),
        scratch_types=[
            pltpu.VMEM((b_per_w,), jnp.int32),
            pltpu.VMEM((b_per_w, _DPAD), jnp.float32),
            pltpu.SemaphoreType.DMA,
        ],
    )
    def k(table_hbm, idx_hbm, out_hbm, idx_v, rows_v, sem):
        wid = lax.axis_index("s") * info.num_cores + lax.axis_index("c")
        base = wid * b_per_w
        pltpu.sync_copy(idx_hbm.at[pl.ds(base, b_per_w)], idx_v)
        pltpu.async_copy(table_hbm.at[idx_v], rows_v, sem).wait()
        pltpu.sync_copy(rows_v, out_hbm.at[pl.ds(base, b_per_w)])

    return k(table, idx)


def kernel(z, emb, W1, b1, W2, b2):
    zp, zpb, embb, table = _zp_and_table(z, emb, W1, b1, W2, b2)
    zpn2 = jnp.sum(zp * zp, axis=1).reshape(_N, 1)
    en3 = jnp.sum(emb * emb, axis=1).reshape(_NVB, 1, _TVB)
    tokens = _tokens(zpb, embb, zpn2, en3).reshape(_N)
    z_q = _gather_rows(table, tokens)[:, :_IN_DIM]
    return tokens, z_q


# speedup vs baseline: 1.5225x; 1.0362x over previous
import functools

import jax
import jax.numpy as jnp
from jax import lax
from jax.experimental import pallas as pl
from jax.experimental.pallas import tpu as pltpu
from jax.experimental.pallas import tpu_sc as plsc

_VOCAB = 8192
_EMBED = 32
_N = 8192
_IN_DIM = 64
_TN = 512
_TV = 512
_NN = _N // _TN
_NV = _VOCAB // _TV
_DPAD = 128


def _a_body(z_ref, w1_ref, b1_ref, emb_ref, w2_ref, b2_ref,
            zp_ref, zpb_ref, embb_ref, table_ref):
    zp = jnp.dot(z_ref[...], w1_ref[...],
                 preferred_element_type=jnp.float32) + b1_ref[...]
    zp_ref[...] = zp
    zpb_ref[...] = zp.astype(jnp.bfloat16)
    embb = emb_ref[...].astype(jnp.bfloat16)
    embb_ref[...] = embb
    table_ref[...] = lax.dot_general(
        embb, w2_ref[...], (((1,), (0,)), ((), ())),
        preferred_element_type=jnp.float32) + b2_ref[...]


def _zp_and_table(z, emb, W1, b1, W2, b2):
    return pl.pallas_call(
        _a_body,
        grid=(_NN,),
        in_specs=[
            pl.BlockSpec((_TN, _IN_DIM), lambda i: (i, 0)),
            pl.BlockSpec((_IN_DIM, _EMBED), lambda i: (0, 0)),
            pl.BlockSpec((1, _EMBED), lambda i: (0, 0)),
            pl.BlockSpec((_TN, _EMBED), lambda i: (i, 0)),
            pl.BlockSpec((_EMBED, _DPAD), lambda i: (0, 0)),
            pl.BlockSpec((1, _DPAD), lambda i: (0, 0)),
        ],
        out_specs=[
            pl.BlockSpec((_TN, _EMBED), lambda i: (i, 0)),
            pl.BlockSpec((_TN, _EMBED), lambda i: (i, 0)),
            pl.BlockSpec((_TN, _EMBED), lambda i: (i, 0)),
            pl.BlockSpec((_TN, _DPAD), lambda i: (i, 0)),
        ],
        out_shape=[
            jax.ShapeDtypeStruct((_N, _EMBED), jnp.float32),
            jax.ShapeDtypeStruct((_N, _EMBED), jnp.bfloat16),
            jax.ShapeDtypeStruct((_VOCAB, _EMBED), jnp.bfloat16),
            jax.ShapeDtypeStruct((_VOCAB, _DPAD), jnp.float32),
        ],
    )(z, W1, b1.reshape(1, _EMBED), emb,
      jnp.pad(W2, ((0, 0), (0, _DPAD - _IN_DIM))),
      jnp.pad(b2, (0, _DPAD - _IN_DIM)).reshape(1, _DPAD))


_TVB = 2048
_TNB = 1024
_NVB = _VOCAB // _TVB


def _b_body(zpb_ref, embb_ref, zpn_ref, en_ref, tok_ref, best_s, besti_s):
    v = pl.program_id(0)
    n = pl.program_id(1)
    nds = pl.ds(pl.multiple_of(n * _TNB, _TNB), _TNB)
    vds = pl.ds(pl.multiple_of(v * _TVB, _TVB), _TVB)

    @pl.when(v == 0)
    def _():
        best_s[nds, :] = jnp.full((_TNB, 1), jnp.inf, dtype=jnp.float32)
        besti_s[nds, :] = jnp.zeros((_TNB, 1), dtype=jnp.int32)

    zpb = zpb_ref[nds, :]
    e = embb_ref[vds, :]
    mm = lax.dot_general(zpb, e, (((1,), (1,)), ((), ())),
                         preferred_element_type=jnp.float32)
    d = (zpn_ref[nds, :] + en_ref[...].reshape(1, _TVB)) - 2.0 * mm
    lmin = jnp.min(d, axis=1).reshape(_TNB, 1)
    cols = lax.broadcasted_iota(jnp.int32, (_TNB, _TVB), 1)
    cand = jnp.where(d == lmin, cols, jnp.int32(0x7FFFFFFF))
    lidx = jnp.min(cand, axis=1).reshape(_TNB, 1)
    upd = lmin < best_s[nds, :]
    besti_s[nds, :] = jnp.where(upd, v * _TVB + lidx, besti_s[nds, :])
    best_s[nds, :] = jnp.where(upd, lmin, best_s[nds, :]).astype(
        jnp.bfloat16).astype(jnp.float32)

    @pl.when(v == _NVB - 1)
    def _():
        tok_ref[...] = besti_s[nds, :]


def _tokens(zpb, embb, zpn2, en3):
    return pl.pallas_call(
        _b_body,
        grid=(_NVB, _N // _TNB),
        in_specs=[
            pl.BlockSpec((_N, _EMBED), lambda v, n: (0, 0)),
            pl.BlockSpec((_VOCAB, _EMBED), lambda v, n: (0, 0)),
            pl.BlockSpec((_N, 1), lambda v, n: (0, 0)),
            pl.BlockSpec((1, 1, _TVB), lambda v, n: (v, 0, 0)),
        ],
        out_specs=pl.BlockSpec((_TNB, 1), lambda v, n: (n, 0)),
        out_shape=jax.ShapeDtypeStruct((_N, 1), jnp.int32),
        scratch_shapes=[
            pltpu.VMEM((_N, 1), jnp.float32),
            pltpu.VMEM((_N, 1), jnp.int32),
        ],
    )(zpb, embb, zpn2, en3)


def _gather_rows(table, idx):
    info = plsc.get_sparse_core_info()
    nw = info.num_cores * info.num_subcores
    b_per_w = _N // nw
    mesh = plsc.VectorSubcoreMesh(core_axis_name="c", subcore_axis_name="s")

    @functools.partial(
        pl.kernel,
        mesh=mesh,
        out_type=jax.ShapeDtypeStruct((_N, _DPAD), jnp.float32),
        scratch_types=[
            pltpu.VMEM((b_per_w,), jnp.int32),
            pltpu.VMEM((b_per_w, _DPAD), jnp.float32),
            pltpu.SemaphoreType.DMA,
        ],
    )
    def k(table_hbm, idx_hbm, out_hbm, idx_v, rows_v, sem):
        wid = lax.axis_index("s") * info.num_cores + lax.axis_index("c")
        base = wid * b_per_w
        pltpu.sync_copy(idx_hbm.at[pl.ds(base, b_per_w)], idx_v)
        pltpu.async_copy(table_hbm.at[idx_v], rows_v, sem).wait()
        pltpu.sync_copy(rows_v, out_hbm.at[pl.ds(base, b_per_w)])

    return k(table, idx)


def kernel(z, emb, W1, b1, W2, b2):
    zp, zpb, embb, table = _zp_and_table(z, emb, W1, b1, W2, b2)
    zpn2 = jnp.sum(zp * zp, axis=1).reshape(_N, 1)
    en3 = jnp.sum(emb * emb, axis=1).reshape(_NVB, 1, _TVB)
    tokens = _tokens(zpb, embb, zpn2, en3).reshape(_N)
    z_q = _gather_rows(table, tokens)[:, :_IN_DIM]
    return tokens, z_q
